# baseline (speedup 1.0000x reference)
import functools

import jax
import jax.numpy as jnp
from jax import lax
from jax.experimental import pallas as pl
from jax.experimental.pallas import tpu as pltpu
from jax.experimental.pallas import tpu_sc as plsc

_NUM_FIELDS = 26
_FIELD_SIZE = 100000
_BATCH = 16384


def _fmlinear(x_flat, tab, perm):
    info = plsc.get_sparse_core_info()
    nw = info.num_cores * info.num_subcores
    lanes = info.num_lanes
    bw = _BATCH // nw
    blk = bw * _NUM_FIELDS

    mesh = plsc.VectorSubcoreMesh(core_axis_name="c", subcore_axis_name="s")

    @functools.partial(
        pl.kernel,
        mesh=mesh,
        out_type=jax.ShapeDtypeStruct((_BATCH,), jnp.float32),
        scratch_types=[
            pltpu.VMEM((blk,), jnp.int32),
            pltpu.VMEM((blk,), jnp.int32),
            pltpu.VMEM((bw,), jnp.int32),
            pltpu.VMEM((bw,), jnp.int32),
            pltpu.VMEM((bw,), jnp.float32),
            pltpu.VMEM((bw,), jnp.float32),
            pltpu.VMEM((bw,), jnp.float32),
            pltpu.SemaphoreType.DMA,
            pltpu.SemaphoreType.DMA,
            pltpu.SemaphoreType.DMA,
        ],
    )
    def k(x_hbm, tab_hbm, perm_hbm, out_hbm, pb, xt,
          idx0, idx1, val0, val1, acc, sem0, sem1, semt):
        wid = lax.axis_index("s") * info.num_cores + lax.axis_index("c")
        base = wid * bw
        bufs = ((idx0, val0, sem0), (idx1, val1, sem1))

        pltpu.sync_copy(perm_hbm, pb)
        xwin = x_hbm.at[pl.ds(base * _NUM_FIELDS, blk)]
        pltpu.async_copy(xwin.at[pb], xt, semt).wait()

        def fire(f, idx_v, val_v, sem):
            off = f * _FIELD_SIZE
            fbase = f * bw
            for c in range(bw // lanes):
                s = pl.ds(c * lanes, lanes)
                idx_v[s] = xt[pl.ds(fbase + c * lanes, lanes)] + off
            return pltpu.async_copy(tab_hbm.at[idx_v], val_v, sem)

        cps = [fire(0, *bufs[0]), fire(1, *bufs[1])]
        for f in range(_NUM_FIELDS):
            p = f % 2
            idx_v, val_v, sem = bufs[p]
            cps[p].wait()
            for c in range(bw // lanes):
                s = pl.ds(c * lanes, lanes)
                if f == 0:
                    acc[s] = val_v[s]
                else:
                    acc[s] = acc[s] + val_v[s]
            if f + 2 < _NUM_FIELDS:
                cps[p] = fire(f + 2, idx_v, val_v, sem)

        pltpu.sync_copy(acc, out_hbm.at[pl.ds(base, bw)])

    return k(x_flat, tab, perm)


def kernel(x, table):
    tab = table.reshape(-1)
    bw = _BATCH // 32
    blk = bw * _NUM_FIELDS
    k = jnp.arange(blk, dtype=jnp.int32)
    perm = (k % bw) * _NUM_FIELDS + k // bw
    out = _fmlinear(x.reshape(-1), tab, perm)
    return out.reshape(_BATCH, 1)


# speedup vs baseline: 1.2539x; 1.2539x over previous
import functools

import jax
import jax.numpy as jnp
from jax import lax
from jax.experimental import pallas as pl
from jax.experimental.pallas import tpu as pltpu
from jax.experimental.pallas import tpu_sc as plsc

_NUM_FIELDS = 26
_FIELD_SIZE = 100000
_BATCH = 16384
_NBUF = 4


def _fmlinear(x_t, tab):
    info = plsc.get_sparse_core_info()
    nw = info.num_cores * info.num_subcores
    lanes = info.num_lanes
    bw = _BATCH // nw

    mesh = plsc.VectorSubcoreMesh(core_axis_name="c", subcore_axis_name="s")

    @functools.partial(
        pl.kernel,
        mesh=mesh,
        out_type=jax.ShapeDtypeStruct((_BATCH,), jnp.float32),
        scratch_types=[
            pltpu.VMEM((_NUM_FIELDS, bw), jnp.int32),
            *([pltpu.VMEM((bw,), jnp.int32)] * _NBUF),
            *([pltpu.VMEM((bw,), jnp.float32)] * _NBUF),
            pltpu.VMEM((bw,), jnp.float32),
            *([pltpu.SemaphoreType.DMA] * _NBUF),
        ],
    )
    def k(x_hbm, tab_hbm, out_hbm, xb,
          i0, i1, i2, i3, v0, v1, v2, v3, acc, s0, s1, s2, s3):
        wid = lax.axis_index("s") * info.num_cores + lax.axis_index("c")
        base = wid * bw
        bufs = ((i0, v0, s0), (i1, v1, s1), (i2, v2, s2), (i3, v3, s3))

        pltpu.sync_copy(x_hbm.at[:, pl.ds(base, bw)], xb)

        def fire(f, idx_v, val_v, sem):
            off = f * _FIELD_SIZE
            for c in range(bw // lanes):
                s = pl.ds(c * lanes, lanes)
                idx_v[s] = xb[f, s] + off
            return pltpu.async_copy(tab_hbm.at[idx_v], val_v, sem)

        cps = [fire(f, *bufs[f]) for f in range(_NBUF)]
        for f in range(_NUM_FIELDS):
            p = f % _NBUF
            idx_v, val_v, sem = bufs[p]
            cps[p].wait()
            for c in range(bw // lanes):
                s = pl.ds(c * lanes, lanes)
                if f == 0:
                    acc[s] = val_v[s]
                else:
                    acc[s] = acc[s] + val_v[s]
            if f + _NBUF < _NUM_FIELDS:
                cps[p] = fire(f + _NBUF, idx_v, val_v, sem)

        pltpu.sync_copy(acc, out_hbm.at[pl.ds(base, bw)])

    return k(x_t, tab)


def kernel(x, table):
    x_t = x.T
    tab = table.reshape(-1)
    out = _fmlinear(x_t, tab)
    return out.reshape(_BATCH, 1)
